# baseline (speedup 1.0000x reference)
import functools

import jax
import jax.numpy as jnp
from jax import lax
from jax.experimental import pallas as pl
from jax.experimental.pallas import tpu as pltpu
from jax.experimental.pallas import tpu_sc as plsc

A_PLUS = 0.04
A_MINUS = -0.04
INV_TAU = 100.0

L = 16
G = 64


@functools.lru_cache(maxsize=None)
def _build(M, H, B):
    info = plsc.get_sparse_core_info()
    NC, NS = info.num_cores, info.num_subcores
    NW = NC * NS
    assert H == L and M % NW == 0 and B % L == 0
    R = M // NW
    npart = 1
    while R % npart != 0 or (R // npart) * H > 50000:
        npart += 1
    CH = R // npart
    NG = B // L

    mesh = plsc.VectorSubcoreMesh(core_axis_name="c", subcore_axis_name="s")

    def body(mem_ref, dt_ref, idx_ref, out_ref,
             chunk, idxb, myev, pev, prow, dtb, sem_in, sem_g):
        wid = lax.axis_index("s") * NC + lax.axis_index("c")
        base = wid * R
        iota = lax.iota(jnp.int32, L)

        pltpu.sync_copy(idx_ref, idxb)

        def scan_body(g, off):
            iv = idxb[pl.ds(g * L, L)]
            m = (iv >= base) & (iv < base + R)
            pos = plsc.cumsum(m.astype(jnp.int32))
            plsc.store_scatter(myev, [off + pos - 1], g * L + iota, mask=m)
            return off + plsc.all_reduce_population_count(m)

        offv = lax.fori_loop(0, NG, scan_body, jnp.zeros((L,), jnp.int32))
        n_my = offv[0]
        ngm = (n_my + (L - 1)) >> 4

        def part_body(p, _):
            pbase = base + p * CH
            cp_in = pltpu.async_copy(
                mem_ref.at[pl.ds(pbase, CH), :], chunk, sem_in)

            def pscan(j, offp):
                valid = (j * L + iota) < n_my
                ev = myev[pl.ds(j * L, L)]
                gi = plsc.load_gather(idxb, [ev], mask=valid)
                pm = valid & (gi >= pbase) & (gi < pbase + CH)
                pos = offp + plsc.cumsum(pm.astype(jnp.int32)) - 1
                plsc.store_scatter(pev, [pos], ev, mask=pm)
                plsc.store_scatter(prow, [pos], gi - pbase, mask=pm)
                return offp + plsc.all_reduce_population_count(pm)

            offpv = lax.fori_loop(0, ngm, pscan, jnp.zeros((L,), jnp.int32))
            n_p = offpv[0]

            zz = jnp.zeros((L,), jnp.int32)

            def padb(k, _):
                pev[pl.ds(n_p + k * L, L)] = zz
                return 0

            lax.fori_loop(0, G // L, padb, 0)
            cp_in.wait()

            nch = (n_p + (G - 1)) >> 6

            def chunk_body(c, _):
                pltpu.async_copy(
                    dt_ref.at[pev.at[pl.ds(c * G, G)]], dtb, sem_g).wait()
                nj = jnp.minimum(n_p - c * G, G)

                def ev_body(j, _):
                    d = dtb[j]
                    pw = plsc.load_gather(
                        prow, [jnp.full((L,), c * G + j, jnp.int32)])
                    dw = jnp.where(
                        d > 0, A_PLUS * jnp.exp(d * (-INV_TAU)),
                        jnp.where(d < 0, A_MINUS * jnp.exp(d * INV_TAU),
                                  jnp.zeros_like(d)))
                    plsc.addupdate_scatter(chunk, [pw, iota], dw)
                    return 0

                lax.fori_loop(0, nj, ev_body, 0)
                return 0

            lax.fori_loop(0, nch, chunk_body, 0)
            pltpu.sync_copy(chunk, out_ref.at[pl.ds(pbase, CH), :])
            return 0

        lax.fori_loop(0, npart, part_body, 0)

    return pl.kernel(
        body,
        out_type=jax.ShapeDtypeStruct((M, H), jnp.float32),
        mesh=mesh,
        compiler_params=pltpu.CompilerParams(
            needs_layout_passes=False, use_tc_tiling_on_sc=False),
        scratch_types=[
            pltpu.VMEM((CH, L), jnp.float32),
            pltpu.VMEM((B,), jnp.int32),
            pltpu.VMEM((B + L,), jnp.int32),
            pltpu.VMEM((B + G,), jnp.int32),
            pltpu.VMEM((B + L,), jnp.int32),
            pltpu.VMEM((G, L), jnp.float32),
            pltpu.SemaphoreType.DMA,
            pltpu.SemaphoreType.DMA,
        ],
    )


def kernel(mem, delta_t, idx):
    M, H = mem.shape
    B = idx.shape[0]
    return _build(M, H, B)(mem, delta_t, idx.astype(jnp.int32))


# speedup vs baseline: 3.0016x; 3.0016x over previous
import functools

import jax
import jax.numpy as jnp
from jax import lax
from jax.experimental import pallas as pl
from jax.experimental.pallas import tpu as pltpu
from jax.experimental.pallas import tpu_sc as plsc

A_PLUS = 0.04
A_MINUS = -0.04
INV_TAU = 100.0

L = 16
G = 32
CB = 32
TB = 2048


def _window(d):
    pos = A_PLUS * jnp.exp(d * (-INV_TAU))
    neg = A_MINUS * jnp.exp(d * INV_TAU)
    return jnp.where(d > 0, pos, jnp.where(d < 0, neg, jnp.zeros_like(d)))


def _dw_body(dt_ref, out_ref):
    out_ref[:, pl.ds(0, L)] = _window(dt_ref[...])


@functools.lru_cache(maxsize=None)
def _build_dw(B, H):
    return pl.pallas_call(
        _dw_body,
        grid=(B // TB,),
        in_specs=[pl.BlockSpec((TB, H), lambda i: (i, 0))],
        out_specs=pl.BlockSpec((TB, 128), lambda i: (i, 0)),
        out_shape=jax.ShapeDtypeStruct((B, 128), jnp.float32),
    )


@functools.lru_cache(maxsize=None)
def _build_sc(M, H, B):
    info = plsc.get_sparse_core_info()
    NC, NS = info.num_cores, info.num_subcores
    NW = NC * NS
    assert H == L and B % L == 0
    NBLK = (M + 127) // 128
    BPW = NBLK // NW
    NCHUNK = (NBLK - (NW - 1) * BPW + CB - 1) // CB
    NG = B // L

    mesh = plsc.VectorSubcoreMesh(core_axis_name="c", subcore_axis_name="s")

    def blk_copy(src, dst, k, cb0, sem):
        return pltpu.make_async_copy(
            src.at[:, pl.ds((cb0 + k) * 128, 128)], dst.at[k], sem)

    def body(mem_ref, dwp_ref, idx_ref, out_ref,
             ch0, ch1, idxb, myev, pev, dtb, sem_in, sem_out, sem_g):
        wid = lax.axis_index("s") * NC + lax.axis_index("c")
        base_blk = wid * BPW
        nblk = jnp.where(wid == NW - 1, NBLK - (NW - 1) * BPW, BPW)
        rbase = base_blk * 128
        rend = (base_blk + nblk) * 128
        iota = lax.iota(jnp.int32, L)

        pltpu.sync_copy(idx_ref, idxb)

        def scan_body(g, off):
            iv = idxb[pl.ds(g * L, L)]
            m = (iv >= rbase) & (iv < rend)
            pos = plsc.cumsum(m.astype(jnp.int32))
            plsc.store_scatter(myev, [off + pos - 1], g * L + iota, mask=m)
            return off + plsc.all_reduce_population_count(m)

        offv = lax.fori_loop(0, NG, scan_body, jnp.zeros((L,), jnp.int32))
        n_my = offv[0]
        ngm = (n_my + (L - 1)) >> 4

        band0 = mem_ref.at[0]
        band1 = mem_ref.at[1]
        oband0 = out_ref.at[0]
        oband1 = out_ref.at[1]

        for c in range(NCHUNK):
            cb0 = base_blk + c * CB
            kb = jnp.minimum(nblk - c * CB, CB)
            crbase = cb0 * 128
            crend = crbase + kb * 128

            def fire_in(k, _):
                blk_copy(band0, ch0, k, cb0, sem_in).start()
                blk_copy(band1, ch1, k, cb0, sem_in).start()
                return 0

            lax.fori_loop(0, kb, fire_in, 0)

            def pscan(j, offp):
                valid = (j * L + iota) < n_my
                ev = myev[pl.ds(j * L, L)]
                gi = plsc.load_gather(idxb, [ev], mask=valid)
                pm = valid & (gi >= crbase) & (gi < crend)
                pos = offp + plsc.cumsum(pm.astype(jnp.int32)) - 1
                plsc.store_scatter(pev, [pos], ev, mask=pm)
                return offp + plsc.all_reduce_population_count(pm)

            offpv = lax.fori_loop(0, ngm, pscan, jnp.zeros((L,), jnp.int32))
            n_p = offpv[0]

            zz = jnp.zeros((L,), jnp.int32)

            def padb(k, _):
                pev[pl.ds(n_p + k * L, L)] = zz
                return 0

            lax.fori_loop(0, G // L, padb, 0)

            def drain_in(k, _):
                blk_copy(band0, ch0, k, cb0, sem_in).wait()
                blk_copy(band1, ch1, k, cb0, sem_in).wait()
                return 0

            lax.fori_loop(0, kb, drain_in, 0)

            nch = (n_p + (G - 1)) // G

            def chunk_body(c2, _):
                pltpu.async_copy(
                    dwp_ref.at[pev.at[pl.ds(c2 * G, G)]], dtb, sem_g).wait()
                nj = jnp.minimum(n_p - c2 * G, G)

                def ev_body(j, _):
                    d = dtb[j, pl.ds(0, L)]
                    ev = plsc.load_gather(
                        pev, [jnp.full((L,), c2 * G + j, jnp.int32)])
                    q = plsc.load_gather(idxb, [ev]) - crbase
                    qb = q >> 7
                    qr = q & 127
                    plsc.addupdate_scatter(
                        ch0, [qb, iota & 7, qr], d, mask=iota < 8)
                    plsc.addupdate_scatter(
                        ch1, [qb, iota & 7, qr], d, mask=iota >= 8)
                    return 0

                lax.fori_loop(0, nj, ev_body, 0)
                return 0

            lax.fori_loop(0, nch, chunk_body, 0)

            def out_copy(src, dst, k):
                return pltpu.make_async_copy(
                    src.at[k], dst.at[:, pl.ds((cb0 + k) * 128, 128)],
                    sem_out)

            def fire_out(k, _):
                out_copy(ch0, oband0, k).start()
                out_copy(ch1, oband1, k).start()
                return 0

            def drain_out(k, _):
                out_copy(ch0, oband0, k).wait()
                out_copy(ch1, oband1, k).wait()
                return 0

            lax.fori_loop(0, kb, fire_out, 0)
            lax.fori_loop(0, kb, drain_out, 0)

    return pl.kernel(
        body,
        out_type=jax.ShapeDtypeStruct((2, 8, M), jnp.float32),
        mesh=mesh,
        compiler_params=pltpu.CompilerParams(
            needs_layout_passes=False, use_tc_tiling_on_sc=True),
        scratch_types=[
            pltpu.VMEM((CB, 8, 128), jnp.float32),
            pltpu.VMEM((CB, 8, 128), jnp.float32),
            pltpu.VMEM((B,), jnp.int32),
            pltpu.VMEM((B + L,), jnp.int32),
            pltpu.VMEM((B + G,), jnp.int32),
            pltpu.VMEM((G, 128), jnp.float32),
            pltpu.SemaphoreType.DMA,
            pltpu.SemaphoreType.DMA,
            pltpu.SemaphoreType.DMA,
        ],
    )


def kernel(mem, delta_t, idx):
    M, H = mem.shape
    B = idx.shape[0]
    dwp = _build_dw(B, H)(delta_t)
    memT = mem.T.reshape(2, 8, M)
    outT = _build_sc(M, H, B)(memT, dwp, idx.astype(jnp.int32))
    return outT.reshape(16, M).T
